# speedup vs baseline: 8.4537x; 8.4537x over previous
import functools

import jax
import jax.numpy as jnp
from jax.experimental import pallas as pl
from jax.experimental.pallas import tpu as pltpu

_PREC = jax.lax.Precision.HIGHEST


def _adj_kernel(edge_ref, w_ref, m_ref, *, N, F, HG):
    e = edge_ref[...]
    src = e[0:1, :]
    dst = e[1:2, :]
    E = e.shape[1]
    nodes = jax.lax.broadcasted_iota(jnp.int32, (N, E), 0)
    maskd = (dst == nodes).astype(jnp.float32)
    masks = (src == nodes).astype(jnp.float32)
    deg = jnp.sum(maskd, axis=1, keepdims=True) + 1.0
    inv = jax.lax.rsqrt(deg)
    count = jax.lax.dot_general(maskd, masks, (((1,), (1,)), ((), ())),
                                preferred_element_type=jnp.float32,
                                precision=_PREC)
    eye = (jax.lax.broadcasted_iota(jnp.int32, (N, N), 0)
           == jax.lax.broadcasted_iota(jnp.int32, (N, N), 1)).astype(jnp.float32)
    outer = jax.lax.dot_general(inv, inv, (((1,), (1,)), ((), ())),
                                preferred_element_type=jnp.float32,
                                precision=_PREC)
    A = (count + eye) * outer

    NF, NH = N * F, N * HG
    R = (jax.lax.broadcasted_iota(jnp.int32, (NF, N), 0) // F
         == jax.lax.broadcasted_iota(jnp.int32, (NF, N), 1)).astype(jnp.float32)
    C = (jax.lax.broadcasted_iota(jnp.int32, (N, NH), 1) // HG
         == jax.lax.broadcasted_iota(jnp.int32, (N, NH), 0)).astype(jnp.float32)
    RA = jax.lax.dot_general(R, A, (((1,), (1,)), ((), ())),
                             preferred_element_type=jnp.float32,
                             precision=_PREC)
    A_big = jax.lax.dot_general(RA, C, (((1,), (0,)), ((), ())),
                                preferred_element_type=jnp.float32,
                                precision=_PREC)
    Rw = (jax.lax.broadcasted_iota(jnp.int32, (NF, F), 0) % F
          == jax.lax.broadcasted_iota(jnp.int32, (NF, F), 1)).astype(jnp.float32)
    Cw = (jax.lax.broadcasted_iota(jnp.int32, (HG, NH), 1) % HG
          == jax.lax.broadcasted_iota(jnp.int32, (HG, NH), 0)).astype(jnp.float32)
    RwW = jax.lax.dot_general(Rw, w_ref[...], (((1,), (0,)), ((), ())),
                              preferred_element_type=jnp.float32,
                              precision=_PREC)
    W_big = jax.lax.dot_general(RwW, Cw, (((1,), (0,)), ((), ())),
                                preferred_element_type=jnp.float32,
                                precision=_PREC)
    m_ref[...] = A_big * W_big


def _main_kernel(x_ref, m_ref, wih_ref, whh_ref, bg_ref, bih_ref, bhh_ref,
                 gcn_ref, h_ref, *, HR):
    t = pl.program_id(0)

    @pl.when(t == 0)
    def _init():
        h_ref[...] = jnp.zeros_like(h_ref)

    xb = x_ref[0]
    gcn = jnp.tanh(
        jax.lax.dot_general(xb, m_ref[...], (((1,), (0,)), ((), ())),
                            preferred_element_type=jnp.float32,
                            precision=_PREC)
        + bg_ref[...])
    gcn_ref[0] = gcn
    gi = jax.lax.dot_general(gcn, wih_ref[...], (((1,), (1,)), ((), ())),
                             preferred_element_type=jnp.float32,
                             precision=_PREC) + bih_ref[...]
    h = h_ref[...]
    gh = jax.lax.dot_general(h, whh_ref[...], (((1,), (1,)), ((), ())),
                             preferred_element_type=jnp.float32,
                             precision=_PREC) + bhh_ref[...]
    r = jax.nn.sigmoid(gi[:, :HR] + gh[:, :HR])
    z = jax.nn.sigmoid(gi[:, HR:2 * HR] + gh[:, HR:2 * HR])
    n = jnp.tanh(gi[:, 2 * HR:] + r * gh[:, 2 * HR:])
    h_ref[...] = (1.0 - z) * n + z * h


def kernel(x, edge_index, W_gcn, b_gcn, W_ih, W_hh, b_ih, b_hh):
    B, T, N, F = x.shape
    HG = W_gcn.shape[1]
    HR = W_hh.shape[1]
    NF, NH = N * F, N * HG

    xT = jnp.transpose(x, (1, 0, 2, 3)).reshape(T, B, NF)

    M = pl.pallas_call(
        functools.partial(_adj_kernel, N=N, F=F, HG=HG),
        out_shape=jax.ShapeDtypeStruct((NF, NH), jnp.float32),
    )(edge_index, W_gcn)

    bg = jnp.tile(b_gcn, N).reshape(1, NH)
    bih = b_ih.reshape(1, 3 * HR)
    bhh = b_hh.reshape(1, 3 * HR)

    gcnT, h_final = pl.pallas_call(
        functools.partial(_main_kernel, HR=HR),
        grid=(T,),
        in_specs=[
            pl.BlockSpec((1, B, NF), lambda t: (t, 0, 0)),
            pl.BlockSpec((NF, NH), lambda t: (0, 0)),
            pl.BlockSpec((3 * HR, NH), lambda t: (0, 0)),
            pl.BlockSpec((3 * HR, HR), lambda t: (0, 0)),
            pl.BlockSpec((1, NH), lambda t: (0, 0)),
            pl.BlockSpec((1, 3 * HR), lambda t: (0, 0)),
            pl.BlockSpec((1, 3 * HR), lambda t: (0, 0)),
        ],
        out_specs=[
            pl.BlockSpec((1, B, NH), lambda t: (t, 0, 0)),
            pl.BlockSpec((B, HR), lambda t: (0, 0)),
        ],
        out_shape=[
            jax.ShapeDtypeStruct((T, B, NH), jnp.float32),
            jax.ShapeDtypeStruct((B, HR), jnp.float32),
        ],
        compiler_params=pltpu.CompilerParams(
            dimension_semantics=("arbitrary",)),
    )(xT, M, W_ih, W_hh, bg, bih, bhh)

    gcn_features = jnp.transpose(gcnT, (1, 0, 2)).reshape(B, T, N, HG)
    return gcn_features, h_final


# speedup vs baseline: 14.3062x; 1.6923x over previous
import functools

import jax
import jax.numpy as jnp
from jax.experimental import pallas as pl
from jax.experimental.pallas import tpu as pltpu

_PREC = jax.lax.Precision.HIGHEST
_MAIN_PREC = jax.lax.Precision.DEFAULT


def _adj_kernel(edge_ref, w_ref, m_ref, *, N, F, HG):
    e = edge_ref[...]
    src = e[0:1, :]
    dst = e[1:2, :]
    E = e.shape[1]
    nodes = jax.lax.broadcasted_iota(jnp.int32, (N, E), 0)
    maskd = (dst == nodes).astype(jnp.float32)
    masks = (src == nodes).astype(jnp.float32)
    deg = jnp.sum(maskd, axis=1, keepdims=True) + 1.0
    inv = jax.lax.rsqrt(deg)
    count = jax.lax.dot_general(maskd, masks, (((1,), (1,)), ((), ())),
                                preferred_element_type=jnp.float32,
                                precision=_PREC)
    eye = (jax.lax.broadcasted_iota(jnp.int32, (N, N), 0)
           == jax.lax.broadcasted_iota(jnp.int32, (N, N), 1)).astype(jnp.float32)
    outer = jax.lax.dot_general(inv, inv, (((1,), (1,)), ((), ())),
                                preferred_element_type=jnp.float32,
                                precision=_PREC)
    A = (count + eye) * outer

    NF, NH = N * F, N * HG
    R = (jax.lax.broadcasted_iota(jnp.int32, (NF, N), 0) // F
         == jax.lax.broadcasted_iota(jnp.int32, (NF, N), 1)).astype(jnp.float32)
    C = (jax.lax.broadcasted_iota(jnp.int32, (N, NH), 1) // HG
         == jax.lax.broadcasted_iota(jnp.int32, (N, NH), 0)).astype(jnp.float32)
    RA = jax.lax.dot_general(R, A, (((1,), (1,)), ((), ())),
                             preferred_element_type=jnp.float32,
                             precision=_PREC)
    A_big = jax.lax.dot_general(RA, C, (((1,), (0,)), ((), ())),
                                preferred_element_type=jnp.float32,
                                precision=_PREC)
    Rw = (jax.lax.broadcasted_iota(jnp.int32, (NF, F), 0) % F
          == jax.lax.broadcasted_iota(jnp.int32, (NF, F), 1)).astype(jnp.float32)
    Cw = (jax.lax.broadcasted_iota(jnp.int32, (HG, NH), 1) % HG
          == jax.lax.broadcasted_iota(jnp.int32, (HG, NH), 0)).astype(jnp.float32)
    RwW = jax.lax.dot_general(Rw, w_ref[...], (((1,), (0,)), ((), ())),
                              preferred_element_type=jnp.float32,
                              precision=_PREC)
    W_big = jax.lax.dot_general(RwW, Cw, (((1,), (0,)), ((), ())),
                                preferred_element_type=jnp.float32,
                                precision=_PREC)
    m_ref[...] = A_big * W_big


def _main_kernel(x_ref, m_ref, wih_ref, whh_ref, bg_ref, bih_ref, bhh_ref,
                 gcn_ref, h_ref, *, HR):
    t = pl.program_id(0)

    @pl.when(t == 0)
    def _init():
        h_ref[...] = jnp.zeros_like(h_ref)

    xb = x_ref[0]
    gcn = jnp.tanh(
        jax.lax.dot_general(xb, m_ref[...], (((1,), (0,)), ((), ())),
                            preferred_element_type=jnp.float32,
                            precision=_MAIN_PREC)
        + bg_ref[...])
    gcn_ref[0] = gcn
    gi = jax.lax.dot_general(gcn, wih_ref[...], (((1,), (1,)), ((), ())),
                             preferred_element_type=jnp.float32,
                             precision=_MAIN_PREC) + bih_ref[...]
    h = h_ref[...]
    gh = jax.lax.dot_general(h, whh_ref[...], (((1,), (1,)), ((), ())),
                             preferred_element_type=jnp.float32,
                             precision=_MAIN_PREC) + bhh_ref[...]
    r = jax.nn.sigmoid(gi[:, :HR] + gh[:, :HR])
    z = jax.nn.sigmoid(gi[:, HR:2 * HR] + gh[:, HR:2 * HR])
    n = jnp.tanh(gi[:, 2 * HR:] + r * gh[:, 2 * HR:])
    h_ref[...] = (1.0 - z) * n + z * h


def kernel(x, edge_index, W_gcn, b_gcn, W_ih, W_hh, b_ih, b_hh):
    B, T, N, F = x.shape
    HG = W_gcn.shape[1]
    HR = W_hh.shape[1]
    NF, NH = N * F, N * HG

    xT = jnp.transpose(x, (1, 0, 2, 3)).reshape(T, B, NF)

    M = pl.pallas_call(
        functools.partial(_adj_kernel, N=N, F=F, HG=HG),
        out_shape=jax.ShapeDtypeStruct((NF, NH), jnp.float32),
    )(edge_index, W_gcn)

    bg = jnp.tile(b_gcn, N).reshape(1, NH)
    bih = b_ih.reshape(1, 3 * HR)
    bhh = b_hh.reshape(1, 3 * HR)

    gcnT, h_final = pl.pallas_call(
        functools.partial(_main_kernel, HR=HR),
        grid=(T,),
        in_specs=[
            pl.BlockSpec((1, B, NF), lambda t: (t, 0, 0)),
            pl.BlockSpec((NF, NH), lambda t: (0, 0)),
            pl.BlockSpec((3 * HR, NH), lambda t: (0, 0)),
            pl.BlockSpec((3 * HR, HR), lambda t: (0, 0)),
            pl.BlockSpec((1, NH), lambda t: (0, 0)),
            pl.BlockSpec((1, 3 * HR), lambda t: (0, 0)),
            pl.BlockSpec((1, 3 * HR), lambda t: (0, 0)),
        ],
        out_specs=[
            pl.BlockSpec((1, B, NH), lambda t: (t, 0, 0)),
            pl.BlockSpec((B, HR), lambda t: (0, 0)),
        ],
        out_shape=[
            jax.ShapeDtypeStruct((T, B, NH), jnp.float32),
            jax.ShapeDtypeStruct((B, HR), jnp.float32),
        ],
        compiler_params=pltpu.CompilerParams(
            dimension_semantics=("arbitrary",)),
    )(xT, M, W_ih, W_hh, bg, bih, bhh)

    gcn_features = jnp.transpose(gcnT, (1, 0, 2)).reshape(B, T, N, HG)
    return gcn_features, h_final


# speedup vs baseline: 19.7189x; 1.3783x over previous
import functools

import jax
import jax.numpy as jnp
from jax.experimental import pallas as pl
from jax.experimental.pallas import tpu as pltpu

_PREC = jax.lax.Precision.HIGHEST
_MAIN_PREC = jax.lax.Precision.DEFAULT


def _adj_kernel(edge_ref, w_ref, m_ref, *, N, F, HG):
    e = edge_ref[...]
    src = e[0:1, :]
    dst = e[1:2, :]
    E = e.shape[1]
    nodes = jax.lax.broadcasted_iota(jnp.int32, (N, E), 0)
    maskd = (dst == nodes).astype(jnp.float32)
    masks = (src == nodes).astype(jnp.float32)
    deg = jnp.sum(maskd, axis=1, keepdims=True) + 1.0
    inv = jax.lax.rsqrt(deg)
    count = jax.lax.dot_general(maskd, masks, (((1,), (1,)), ((), ())),
                                preferred_element_type=jnp.float32,
                                precision=_PREC)
    eye = (jax.lax.broadcasted_iota(jnp.int32, (N, N), 0)
           == jax.lax.broadcasted_iota(jnp.int32, (N, N), 1)).astype(jnp.float32)
    outer = jax.lax.dot_general(inv, inv, (((1,), (1,)), ((), ())),
                                preferred_element_type=jnp.float32,
                                precision=_PREC)
    A = (count + eye) * outer

    NF, NH = N * F, N * HG
    R = (jax.lax.broadcasted_iota(jnp.int32, (NF, N), 0) // F
         == jax.lax.broadcasted_iota(jnp.int32, (NF, N), 1)).astype(jnp.float32)
    C = (jax.lax.broadcasted_iota(jnp.int32, (N, NH), 1) // HG
         == jax.lax.broadcasted_iota(jnp.int32, (N, NH), 0)).astype(jnp.float32)
    RA = jax.lax.dot_general(R, A, (((1,), (1,)), ((), ())),
                             preferred_element_type=jnp.float32,
                             precision=_PREC)
    A_big = jax.lax.dot_general(RA, C, (((1,), (0,)), ((), ())),
                                preferred_element_type=jnp.float32,
                                precision=_PREC)
    Rw = (jax.lax.broadcasted_iota(jnp.int32, (NF, F), 0) % F
          == jax.lax.broadcasted_iota(jnp.int32, (NF, F), 1)).astype(jnp.float32)
    Cw = (jax.lax.broadcasted_iota(jnp.int32, (HG, NH), 1) % HG
          == jax.lax.broadcasted_iota(jnp.int32, (HG, NH), 0)).astype(jnp.float32)
    RwW = jax.lax.dot_general(Rw, w_ref[...], (((1,), (0,)), ((), ())),
                              preferred_element_type=jnp.float32,
                              precision=_PREC)
    W_big = jax.lax.dot_general(RwW, Cw, (((1,), (0,)), ((), ())),
                                preferred_element_type=jnp.float32,
                                precision=_PREC)
    m_ref[...] = A_big * W_big


def _main_kernel(x_ref, m_ref, wih_ref, whh_ref, bg_ref, bih_ref, bhh_ref,
                 gcn_ref, h_ref, *, HR, TCH):
    tb = pl.program_id(0)

    @pl.when(tb == 0)
    def _init():
        h_ref[...] = jnp.zeros_like(h_ref)

    h = h_ref[...]
    for i in range(TCH):
        xb = x_ref[:, i, :]
        gcn = jnp.tanh(
            jax.lax.dot_general(xb, m_ref[...], (((1,), (0,)), ((), ())),
                                preferred_element_type=jnp.float32,
                                precision=_MAIN_PREC)
            + bg_ref[...])
        gcn_ref[:, i, :] = gcn
        gi = jax.lax.dot_general(gcn, wih_ref[...], (((1,), (1,)), ((), ())),
                                 preferred_element_type=jnp.float32,
                                 precision=_MAIN_PREC) + bih_ref[...]
        gh = jax.lax.dot_general(h, whh_ref[...], (((1,), (1,)), ((), ())),
                                 preferred_element_type=jnp.float32,
                                 precision=_MAIN_PREC) + bhh_ref[...]
        r = jax.nn.sigmoid(gi[:, :HR] + gh[:, :HR])
        z = jax.nn.sigmoid(gi[:, HR:2 * HR] + gh[:, HR:2 * HR])
        n = jnp.tanh(gi[:, 2 * HR:] + r * gh[:, 2 * HR:])
        h = (1.0 - z) * n + z * h
    h_ref[...] = h


def kernel(x, edge_index, W_gcn, b_gcn, W_ih, W_hh, b_ih, b_hh):
    B, T, N, F = x.shape
    HG = W_gcn.shape[1]
    HR = W_hh.shape[1]
    NF, NH = N * F, N * HG

    xf = x.reshape(B, T, NF)

    M = pl.pallas_call(
        functools.partial(_adj_kernel, N=N, F=F, HG=HG),
        out_shape=jax.ShapeDtypeStruct((NF, NH), jnp.float32),
    )(edge_index, W_gcn)

    bg = jnp.tile(b_gcn, N).reshape(1, NH)
    bih = b_ih.reshape(1, 3 * HR)
    bhh = b_hh.reshape(1, 3 * HR)

    TCH = 8
    assert T % TCH == 0
    gcnBT, h_final = pl.pallas_call(
        functools.partial(_main_kernel, HR=HR, TCH=TCH),
        grid=(T // TCH,),
        in_specs=[
            pl.BlockSpec((B, TCH, NF), lambda t: (0, t, 0)),
            pl.BlockSpec((NF, NH), lambda t: (0, 0)),
            pl.BlockSpec((3 * HR, NH), lambda t: (0, 0)),
            pl.BlockSpec((3 * HR, HR), lambda t: (0, 0)),
            pl.BlockSpec((1, NH), lambda t: (0, 0)),
            pl.BlockSpec((1, 3 * HR), lambda t: (0, 0)),
            pl.BlockSpec((1, 3 * HR), lambda t: (0, 0)),
        ],
        out_specs=[
            pl.BlockSpec((B, TCH, NH), lambda t: (0, t, 0)),
            pl.BlockSpec((B, HR), lambda t: (0, 0)),
        ],
        out_shape=[
            jax.ShapeDtypeStruct((B, T, NH), jnp.float32),
            jax.ShapeDtypeStruct((B, HR), jnp.float32),
        ],
        compiler_params=pltpu.CompilerParams(
            dimension_semantics=("arbitrary",)),
    )(xf, M, W_ih, W_hh, bg, bih, bhh)

    gcn_features = gcnBT.reshape(B, T, N, HG)
    return gcn_features, h_final


# speedup vs baseline: 22.6120x; 1.1467x over previous
import functools

import jax
import jax.numpy as jnp
from jax.experimental import pallas as pl
from jax.experimental.pallas import tpu as pltpu

_PREC = jax.lax.Precision.HIGHEST
_MAIN_PREC = jax.lax.Precision.DEFAULT


def _adj_kernel(edge_ref, w_ref, m_ref, *, N, F, HG):
    e = edge_ref[...]
    src = e[0:1, :]
    dst = e[1:2, :]
    E = e.shape[1]
    nodes = jax.lax.broadcasted_iota(jnp.int32, (N, E), 0)
    maskd = (dst == nodes).astype(jnp.float32)
    masks = (src == nodes).astype(jnp.float32)
    deg = jnp.sum(maskd, axis=1, keepdims=True) + 1.0
    inv = jax.lax.rsqrt(deg)
    count = jax.lax.dot_general(maskd, masks, (((1,), (1,)), ((), ())),
                                preferred_element_type=jnp.float32,
                                precision=_PREC)
    eye = (jax.lax.broadcasted_iota(jnp.int32, (N, N), 0)
           == jax.lax.broadcasted_iota(jnp.int32, (N, N), 1)).astype(jnp.float32)
    outer = jax.lax.dot_general(inv, inv, (((1,), (1,)), ((), ())),
                                preferred_element_type=jnp.float32,
                                precision=_PREC)
    A = (count + eye) * outer

    NF, NH = N * F, N * HG
    R = (jax.lax.broadcasted_iota(jnp.int32, (NF, N), 0) // F
         == jax.lax.broadcasted_iota(jnp.int32, (NF, N), 1)).astype(jnp.float32)
    C = (jax.lax.broadcasted_iota(jnp.int32, (N, NH), 1) // HG
         == jax.lax.broadcasted_iota(jnp.int32, (N, NH), 0)).astype(jnp.float32)
    RA = jax.lax.dot_general(R, A, (((1,), (1,)), ((), ())),
                             preferred_element_type=jnp.float32,
                             precision=_PREC)
    A_big = jax.lax.dot_general(RA, C, (((1,), (0,)), ((), ())),
                                preferred_element_type=jnp.float32,
                                precision=_PREC)
    Rw = (jax.lax.broadcasted_iota(jnp.int32, (NF, F), 0) % F
          == jax.lax.broadcasted_iota(jnp.int32, (NF, F), 1)).astype(jnp.float32)
    Cw = (jax.lax.broadcasted_iota(jnp.int32, (HG, NH), 1) % HG
          == jax.lax.broadcasted_iota(jnp.int32, (HG, NH), 0)).astype(jnp.float32)
    RwW = jax.lax.dot_general(Rw, w_ref[...], (((1,), (0,)), ((), ())),
                              preferred_element_type=jnp.float32,
                              precision=_PREC)
    W_big = jax.lax.dot_general(RwW, Cw, (((1,), (0,)), ((), ())),
                                preferred_element_type=jnp.float32,
                                precision=_PREC)
    m_ref[...] = A_big * W_big


def _main_kernel(edge_ref, w_ref, x_ref, wih_ref, whh_ref, bg_ref, bih_ref,
                 bhh_ref, gcn_ref, h_ref, m_ref, *, N, F, HG, HR, TCH):
    tb = pl.program_id(0)

    @pl.when(tb == 0)
    def _init():
        h_ref[...] = jnp.zeros_like(h_ref)
        _adj_kernel(edge_ref, w_ref, m_ref, N=N, F=F, HG=HG)

    B = x_ref.shape[0]
    NH = m_ref.shape[1]
    xall = x_ref[...].reshape(B * TCH, x_ref.shape[2])
    gcn_all = jnp.tanh(
        jax.lax.dot_general(xall, m_ref[...], (((1,), (0,)), ((), ())),
                            preferred_element_type=jnp.float32,
                            precision=_MAIN_PREC)
        + bg_ref[...])
    gcn_ref[...] = gcn_all.reshape(B, TCH, NH)
    gi_all = jax.lax.dot_general(gcn_all.astype(jnp.bfloat16),
                                 wih_ref[...].astype(jnp.bfloat16),
                                 (((1,), (1,)), ((), ())),
                                 preferred_element_type=jnp.float32,
                                 precision=_MAIN_PREC) + bih_ref[...]
    gi_tr = jnp.transpose(gi_all.reshape(B, TCH, 3 * HR), (1, 0, 2))
    h = h_ref[...]
    for i in range(TCH):
        gi = gi_tr[i]
        gh = jax.lax.dot_general(h, whh_ref[...], (((1,), (1,)), ((), ())),
                                 preferred_element_type=jnp.float32,
                                 precision=_MAIN_PREC) + bhh_ref[...]
        r = jax.nn.sigmoid(gi[:, :HR] + gh[:, :HR])
        z = jax.nn.sigmoid(gi[:, HR:2 * HR] + gh[:, HR:2 * HR])
        n = jnp.tanh(gi[:, 2 * HR:] + r * gh[:, 2 * HR:])
        h = (1.0 - z) * n + z * h
    h_ref[...] = h


def kernel(x, edge_index, W_gcn, b_gcn, W_ih, W_hh, b_ih, b_hh):
    B, T, N, F = x.shape
    HG = W_gcn.shape[1]
    HR = W_hh.shape[1]
    NF, NH = N * F, N * HG

    xf = x.reshape(B, T, NF)

    bg = jnp.tile(b_gcn, N).reshape(1, NH)
    bih = b_ih.reshape(1, 3 * HR)
    bhh = b_hh.reshape(1, 3 * HR)

    TCH = 8
    assert T % TCH == 0
    gcnBT, h_final = pl.pallas_call(
        functools.partial(_main_kernel, N=N, F=F, HG=HG, HR=HR, TCH=TCH),
        grid=(T // TCH,),
        in_specs=[
            pl.BlockSpec((2, edge_index.shape[1]), lambda t: (0, 0)),
            pl.BlockSpec((F, HG), lambda t: (0, 0)),
            pl.BlockSpec((B, TCH, NF), lambda t: (0, t, 0)),
            pl.BlockSpec((3 * HR, NH), lambda t: (0, 0)),
            pl.BlockSpec((3 * HR, HR), lambda t: (0, 0)),
            pl.BlockSpec((1, NH), lambda t: (0, 0)),
            pl.BlockSpec((1, 3 * HR), lambda t: (0, 0)),
            pl.BlockSpec((1, 3 * HR), lambda t: (0, 0)),
        ],
        out_specs=[
            pl.BlockSpec((B, TCH, NH), lambda t: (0, t, 0)),
            pl.BlockSpec((B, HR), lambda t: (0, 0)),
        ],
        out_shape=[
            jax.ShapeDtypeStruct((B, T, NH), jnp.float32),
            jax.ShapeDtypeStruct((B, HR), jnp.float32),
        ],
        scratch_shapes=[pltpu.VMEM((NF, NH), jnp.float32)],
        compiler_params=pltpu.CompilerParams(
            dimension_semantics=("arbitrary",)),
    )(edge_index, W_gcn, xf, W_ih, W_hh, bg, bih, bhh)

    gcn_features = gcnBT.reshape(B, T, N, HG)
    return gcn_features, h_final


# speedup vs baseline: 23.7764x; 1.0515x over previous
import functools

import jax
import jax.numpy as jnp
from jax.experimental import pallas as pl
from jax.experimental.pallas import tpu as pltpu

_PREC = jax.lax.Precision.HIGHEST
_MAIN_PREC = jax.lax.Precision.DEFAULT


def _adj_kernel(edge_ref, w_ref, m_ref, *, N, F, HG):
    e = edge_ref[...]
    src = e[0:1, :]
    dst = e[1:2, :]
    E = e.shape[1]
    nodes = jax.lax.broadcasted_iota(jnp.int32, (N, E), 0)
    maskd = (dst == nodes).astype(jnp.float32)
    masks = (src == nodes).astype(jnp.float32)
    deg = jnp.sum(maskd, axis=1, keepdims=True) + 1.0
    inv = jax.lax.rsqrt(deg)
    count = jax.lax.dot_general(maskd, masks, (((1,), (1,)), ((), ())),
                                preferred_element_type=jnp.float32,
                                precision=_PREC)
    eye = (jax.lax.broadcasted_iota(jnp.int32, (N, N), 0)
           == jax.lax.broadcasted_iota(jnp.int32, (N, N), 1)).astype(jnp.float32)
    outer = jax.lax.dot_general(inv, inv, (((1,), (1,)), ((), ())),
                                preferred_element_type=jnp.float32,
                                precision=_PREC)
    A = (count + eye) * outer

    NF, NH = N * F, N * HG
    R = (jax.lax.broadcasted_iota(jnp.int32, (NF, N), 0) // F
         == jax.lax.broadcasted_iota(jnp.int32, (NF, N), 1)).astype(jnp.float32)
    C = (jax.lax.broadcasted_iota(jnp.int32, (N, NH), 1) // HG
         == jax.lax.broadcasted_iota(jnp.int32, (N, NH), 0)).astype(jnp.float32)
    RA = jax.lax.dot_general(R, A, (((1,), (1,)), ((), ())),
                             preferred_element_type=jnp.float32,
                             precision=_PREC)
    A_big = jax.lax.dot_general(RA, C, (((1,), (0,)), ((), ())),
                                preferred_element_type=jnp.float32,
                                precision=_PREC)
    Rw = (jax.lax.broadcasted_iota(jnp.int32, (NF, F), 0) % F
          == jax.lax.broadcasted_iota(jnp.int32, (NF, F), 1)).astype(jnp.float32)
    Cw = (jax.lax.broadcasted_iota(jnp.int32, (HG, NH), 1) % HG
          == jax.lax.broadcasted_iota(jnp.int32, (HG, NH), 0)).astype(jnp.float32)
    RwW = jax.lax.dot_general(Rw, w_ref[...], (((1,), (0,)), ((), ())),
                              preferred_element_type=jnp.float32,
                              precision=_PREC)
    W_big = jax.lax.dot_general(RwW, Cw, (((1,), (0,)), ((), ())),
                                preferred_element_type=jnp.float32,
                                precision=_PREC)
    m_ref[...] = A_big * W_big


def _main_kernel(edge_ref, w_ref, x_ref, wih_ref, whh_ref, bg_ref, bih_ref,
                 bhh_ref, gcn_ref, h_ref, m_ref, *, N, F, HG, HR, TCH):
    tb = pl.program_id(0)

    @pl.when(tb == 0)
    def _init():
        h_ref[...] = jnp.zeros_like(h_ref)
        _adj_kernel(edge_ref, w_ref, m_ref, N=N, F=F, HG=HG)

    B = x_ref.shape[0]
    NH = m_ref.shape[1]
    xall = x_ref[...].reshape(B * TCH, x_ref.shape[2])
    gcn_all = jnp.tanh(
        jax.lax.dot_general(xall.astype(jnp.bfloat16),
                            m_ref[...].astype(jnp.bfloat16),
                            (((1,), (0,)), ((), ())),
                            preferred_element_type=jnp.float32,
                            precision=_MAIN_PREC)
        + bg_ref[...])
    gcn_ref[...] = gcn_all.astype(jnp.bfloat16).reshape(B, TCH, NH)
    gi_all = jax.lax.dot_general(gcn_all.astype(jnp.bfloat16),
                                 wih_ref[...].astype(jnp.bfloat16),
                                 (((1,), (1,)), ((), ())),
                                 preferred_element_type=jnp.float32,
                                 precision=_MAIN_PREC) + bih_ref[...]
    gi_tr = jnp.transpose(gi_all.reshape(B, TCH, 3 * HR), (1, 0, 2))
    h = h_ref[...]
    for i in range(TCH):
        gi = gi_tr[i]
        gh = jax.lax.dot_general(h, whh_ref[...], (((1,), (1,)), ((), ())),
                                 preferred_element_type=jnp.float32,
                                 precision=_MAIN_PREC) + bhh_ref[...]
        r = jax.nn.sigmoid(gi[:, :HR] + gh[:, :HR])
        z = jax.nn.sigmoid(gi[:, HR:2 * HR] + gh[:, HR:2 * HR])
        n = jnp.tanh(gi[:, 2 * HR:] + r * gh[:, 2 * HR:])
        h = (1.0 - z) * n + z * h
    h_ref[...] = h


def kernel(x, edge_index, W_gcn, b_gcn, W_ih, W_hh, b_ih, b_hh):
    B, T, N, F = x.shape
    HG = W_gcn.shape[1]
    HR = W_hh.shape[1]
    NF, NH = N * F, N * HG

    xf = x.reshape(B, T, NF)

    bg = jnp.tile(b_gcn, N).reshape(1, NH)
    bih = b_ih.reshape(1, 3 * HR)
    bhh = b_hh.reshape(1, 3 * HR)

    TCH = 8
    assert T % TCH == 0
    gcnBT, h_final = pl.pallas_call(
        functools.partial(_main_kernel, N=N, F=F, HG=HG, HR=HR, TCH=TCH),
        grid=(T // TCH,),
        in_specs=[
            pl.BlockSpec((2, edge_index.shape[1]), lambda t: (0, 0)),
            pl.BlockSpec((F, HG), lambda t: (0, 0)),
            pl.BlockSpec((B, TCH, NF), lambda t: (0, t, 0)),
            pl.BlockSpec((3 * HR, NH), lambda t: (0, 0)),
            pl.BlockSpec((3 * HR, HR), lambda t: (0, 0)),
            pl.BlockSpec((1, NH), lambda t: (0, 0)),
            pl.BlockSpec((1, 3 * HR), lambda t: (0, 0)),
            pl.BlockSpec((1, 3 * HR), lambda t: (0, 0)),
        ],
        out_specs=[
            pl.BlockSpec((B, TCH, NH), lambda t: (0, t, 0)),
            pl.BlockSpec((B, HR), lambda t: (0, 0)),
        ],
        out_shape=[
            jax.ShapeDtypeStruct((B, T, NH), jnp.bfloat16),
            jax.ShapeDtypeStruct((B, HR), jnp.float32),
        ],
        scratch_shapes=[pltpu.VMEM((NF, NH), jnp.float32)],
        compiler_params=pltpu.CompilerParams(
            dimension_semantics=("arbitrary",)),
    )(edge_index, W_gcn, xf, W_ih, W_hh, bg, bih, bhh)

    gcn_features = gcnBT.reshape(B, T, N, HG).astype(jnp.float32)
    return gcn_features, h_final


# speedup vs baseline: 23.9053x; 1.0054x over previous
import functools

import jax
import jax.numpy as jnp
from jax.experimental import pallas as pl
from jax.experimental.pallas import tpu as pltpu

_PREC = jax.lax.Precision.HIGHEST
_MAIN_PREC = jax.lax.Precision.DEFAULT


def _adj_kernel(edge_ref, w_ref, m_ref, *, N, F, HG):
    e = edge_ref[...]
    src = e[0:1, :]
    dst = e[1:2, :]
    E = e.shape[1]
    nodes = jax.lax.broadcasted_iota(jnp.int32, (N, E), 0)
    maskd = (dst == nodes).astype(jnp.float32)
    masks = (src == nodes).astype(jnp.float32)
    deg = jnp.sum(maskd, axis=1, keepdims=True) + 1.0
    inv = jax.lax.rsqrt(deg)
    count = jax.lax.dot_general(maskd, masks, (((1,), (1,)), ((), ())),
                                preferred_element_type=jnp.float32,
                                precision=_PREC)
    eye = (jax.lax.broadcasted_iota(jnp.int32, (N, N), 0)
           == jax.lax.broadcasted_iota(jnp.int32, (N, N), 1)).astype(jnp.float32)
    outer = jax.lax.dot_general(inv, inv, (((1,), (1,)), ((), ())),
                                preferred_element_type=jnp.float32,
                                precision=_PREC)
    A = (count + eye) * outer

    NF, NH = N * F, N * HG
    R = (jax.lax.broadcasted_iota(jnp.int32, (NF, N), 0) // F
         == jax.lax.broadcasted_iota(jnp.int32, (NF, N), 1)).astype(jnp.float32)
    C = (jax.lax.broadcasted_iota(jnp.int32, (N, NH), 1) // HG
         == jax.lax.broadcasted_iota(jnp.int32, (N, NH), 0)).astype(jnp.float32)
    RA = jax.lax.dot_general(R, A, (((1,), (1,)), ((), ())),
                             preferred_element_type=jnp.float32,
                             precision=_PREC)
    A_big = jax.lax.dot_general(RA, C, (((1,), (0,)), ((), ())),
                                preferred_element_type=jnp.float32,
                                precision=_PREC)
    Rw = (jax.lax.broadcasted_iota(jnp.int32, (NF, F), 0) % F
          == jax.lax.broadcasted_iota(jnp.int32, (NF, F), 1)).astype(jnp.float32)
    Cw = (jax.lax.broadcasted_iota(jnp.int32, (HG, NH), 1) % HG
          == jax.lax.broadcasted_iota(jnp.int32, (HG, NH), 0)).astype(jnp.float32)
    RwW = jax.lax.dot_general(Rw, w_ref[...], (((1,), (0,)), ((), ())),
                              preferred_element_type=jnp.float32,
                              precision=_PREC)
    W_big = jax.lax.dot_general(RwW, Cw, (((1,), (0,)), ((), ())),
                                preferred_element_type=jnp.float32,
                                precision=_PREC)
    m_ref[...] = A_big * W_big


def _main_kernel(edge_ref, w_ref, x_ref, wih_ref, whh_ref, bg_ref, bih_ref,
                 bhh_ref, gcn_ref, h_ref, m_ref, *, N, F, HG, HR, TCH):
    tb = pl.program_id(0)

    @pl.when(tb == 0)
    def _init():
        h_ref[...] = jnp.zeros_like(h_ref)
        _adj_kernel(edge_ref, w_ref, m_ref, N=N, F=F, HG=HG)

    B = x_ref.shape[0]
    NH = m_ref.shape[1]
    xall = x_ref[...].reshape(B * TCH, x_ref.shape[2])
    gcn_all = jnp.tanh(
        jax.lax.dot_general(xall,
                            m_ref[...].astype(jnp.bfloat16),
                            (((1,), (0,)), ((), ())),
                            preferred_element_type=jnp.float32,
                            precision=_MAIN_PREC)
        + bg_ref[...])
    gcn_ref[...] = gcn_all.astype(jnp.bfloat16).reshape(B, TCH, NH)
    gi_all = jax.lax.dot_general(gcn_all.astype(jnp.bfloat16),
                                 wih_ref[...].astype(jnp.bfloat16),
                                 (((1,), (1,)), ((), ())),
                                 preferred_element_type=jnp.float32,
                                 precision=_MAIN_PREC) + bih_ref[...]
    gi_tr = jnp.transpose(gi_all.reshape(B, TCH, 3 * HR), (1, 0, 2))
    h = h_ref[...]
    for i in range(TCH):
        gi = gi_tr[i]
        gh = jax.lax.dot_general(h, whh_ref[...], (((1,), (1,)), ((), ())),
                                 preferred_element_type=jnp.float32,
                                 precision=_MAIN_PREC) + bhh_ref[...]
        r = jax.nn.sigmoid(gi[:, :HR] + gh[:, :HR])
        z = jax.nn.sigmoid(gi[:, HR:2 * HR] + gh[:, HR:2 * HR])
        n = jnp.tanh(gi[:, 2 * HR:] + r * gh[:, 2 * HR:])
        h = (1.0 - z) * n + z * h
    h_ref[...] = h


def kernel(x, edge_index, W_gcn, b_gcn, W_ih, W_hh, b_ih, b_hh):
    B, T, N, F = x.shape
    HG = W_gcn.shape[1]
    HR = W_hh.shape[1]
    NF, NH = N * F, N * HG

    xf = x.astype(jnp.bfloat16).reshape(B, T, NF)

    bg = jnp.tile(b_gcn, N).reshape(1, NH)
    bih = b_ih.reshape(1, 3 * HR)
    bhh = b_hh.reshape(1, 3 * HR)

    TCH = 8
    assert T % TCH == 0
    gcnBT, h_final = pl.pallas_call(
        functools.partial(_main_kernel, N=N, F=F, HG=HG, HR=HR, TCH=TCH),
        grid=(T // TCH,),
        in_specs=[
            pl.BlockSpec((2, edge_index.shape[1]), lambda t: (0, 0)),
            pl.BlockSpec((F, HG), lambda t: (0, 0)),
            pl.BlockSpec((B, TCH, NF), lambda t: (0, t, 0)),
            pl.BlockSpec((3 * HR, NH), lambda t: (0, 0)),
            pl.BlockSpec((3 * HR, HR), lambda t: (0, 0)),
            pl.BlockSpec((1, NH), lambda t: (0, 0)),
            pl.BlockSpec((1, 3 * HR), lambda t: (0, 0)),
            pl.BlockSpec((1, 3 * HR), lambda t: (0, 0)),
        ],
        out_specs=[
            pl.BlockSpec((B, TCH, NH), lambda t: (0, t, 0)),
            pl.BlockSpec((B, HR), lambda t: (0, 0)),
        ],
        out_shape=[
            jax.ShapeDtypeStruct((B, T, NH), jnp.bfloat16),
            jax.ShapeDtypeStruct((B, HR), jnp.float32),
        ],
        scratch_shapes=[pltpu.VMEM((NF, NH), jnp.float32)],
        compiler_params=pltpu.CompilerParams(
            dimension_semantics=("arbitrary",)),
    )(edge_index, W_gcn, xf, W_ih, W_hh, bg, bih, bhh)

    gcn_features = gcnBT.reshape(B, T, N, HG).astype(jnp.float32)
    return gcn_features, h_final


# speedup vs baseline: 24.3606x; 1.0190x over previous
import functools

import jax
import jax.numpy as jnp
from jax.experimental import pallas as pl
from jax.experimental.pallas import tpu as pltpu

_PREC = jax.lax.Precision.HIGHEST
_MAIN_PREC = jax.lax.Precision.DEFAULT


def _adj_kernel(edge_ref, w_ref, m_ref, *, N, F, HG):
    e = edge_ref[...]
    src = e[0:1, :]
    dst = e[1:2, :]
    E = e.shape[1]
    nodes = jax.lax.broadcasted_iota(jnp.int32, (N, E), 0)
    maskd = (dst == nodes).astype(jnp.float32)
    masks = (src == nodes).astype(jnp.float32)
    deg = jnp.sum(maskd, axis=1, keepdims=True) + 1.0
    inv = jax.lax.rsqrt(deg)
    count = jax.lax.dot_general(maskd, masks, (((1,), (1,)), ((), ())),
                                preferred_element_type=jnp.float32,
                                precision=_PREC)
    eye = (jax.lax.broadcasted_iota(jnp.int32, (N, N), 0)
           == jax.lax.broadcasted_iota(jnp.int32, (N, N), 1)).astype(jnp.float32)
    outer = jax.lax.dot_general(inv, inv, (((1,), (1,)), ((), ())),
                                preferred_element_type=jnp.float32,
                                precision=_PREC)
    A = (count + eye) * outer

    NF, NH = N * F, N * HG
    R = (jax.lax.broadcasted_iota(jnp.int32, (NF, N), 0) // F
         == jax.lax.broadcasted_iota(jnp.int32, (NF, N), 1)).astype(jnp.float32)
    C = (jax.lax.broadcasted_iota(jnp.int32, (N, NH), 1) // HG
         == jax.lax.broadcasted_iota(jnp.int32, (N, NH), 0)).astype(jnp.float32)
    RA = jax.lax.dot_general(R, A, (((1,), (1,)), ((), ())),
                             preferred_element_type=jnp.float32,
                             precision=_PREC)
    A_big = jax.lax.dot_general(RA, C, (((1,), (0,)), ((), ())),
                                preferred_element_type=jnp.float32,
                                precision=_PREC)
    Rw = (jax.lax.broadcasted_iota(jnp.int32, (NF, F), 0) % F
          == jax.lax.broadcasted_iota(jnp.int32, (NF, F), 1)).astype(jnp.float32)
    Cw = (jax.lax.broadcasted_iota(jnp.int32, (HG, NH), 1) % HG
          == jax.lax.broadcasted_iota(jnp.int32, (HG, NH), 0)).astype(jnp.float32)
    RwW = jax.lax.dot_general(Rw, w_ref[...], (((1,), (0,)), ((), ())),
                              preferred_element_type=jnp.float32,
                              precision=_PREC)
    W_big = jax.lax.dot_general(RwW, Cw, (((1,), (0,)), ((), ())),
                                preferred_element_type=jnp.float32,
                                precision=_PREC)
    m_ref[...] = A_big * W_big


def _main_kernel(edge_ref, w_ref, x_ref, wih_ref, whh_ref, bg_ref, bih_ref,
                 bhh_ref, gcn_ref, h_ref, m_ref, *, N, F, HG, HR, TCH):
    tb = pl.program_id(0)

    @pl.when(tb == 0)
    def _init():
        h_ref[...] = jnp.zeros_like(h_ref)
        _adj_kernel(edge_ref, w_ref, m_ref, N=N, F=F, HG=HG)

    B = x_ref.shape[0]
    NH = m_ref.shape[1]
    xall = x_ref[...].reshape(B * TCH, x_ref.shape[2])
    gcn_all = jnp.tanh(
        jax.lax.dot_general(xall,
                            m_ref[...].astype(jnp.bfloat16),
                            (((1,), (0,)), ((), ())),
                            preferred_element_type=jnp.float32,
                            precision=_MAIN_PREC)
        + bg_ref[...])
    gcn_ref[...] = gcn_all.astype(jnp.bfloat16).reshape(B, TCH, NH)
    gi_all = jax.lax.dot_general(gcn_all.astype(jnp.bfloat16),
                                 wih_ref[...].astype(jnp.bfloat16),
                                 (((1,), (1,)), ((), ())),
                                 preferred_element_type=jnp.float32,
                                 precision=_MAIN_PREC).astype(jnp.bfloat16)
    gi_tr = jnp.transpose(gi_all.reshape(B, TCH, 3 * HR), (1, 0, 2))
    h = h_ref[...]
    for i in range(TCH):
        gi = gi_tr[i].astype(jnp.float32) + bih_ref[...]
        gh = jax.lax.dot_general(h, whh_ref[...], (((1,), (1,)), ((), ())),
                                 preferred_element_type=jnp.float32,
                                 precision=_MAIN_PREC) + bhh_ref[...]
        r = jax.nn.sigmoid(gi[:, :HR] + gh[:, :HR])
        z = jax.nn.sigmoid(gi[:, HR:2 * HR] + gh[:, HR:2 * HR])
        n = jnp.tanh(gi[:, 2 * HR:] + r * gh[:, 2 * HR:])
        h = (1.0 - z) * n + z * h
    h_ref[...] = h


def kernel(x, edge_index, W_gcn, b_gcn, W_ih, W_hh, b_ih, b_hh):
    B, T, N, F = x.shape
    HG = W_gcn.shape[1]
    HR = W_hh.shape[1]
    NF, NH = N * F, N * HG

    xf = x.astype(jnp.bfloat16).reshape(B, T, NF)

    bg = jnp.tile(b_gcn, N).reshape(1, NH)
    bih = b_ih.reshape(1, 3 * HR)
    bhh = b_hh.reshape(1, 3 * HR)

    TCH = 8
    assert T % TCH == 0
    gcnBT, h_final = pl.pallas_call(
        functools.partial(_main_kernel, N=N, F=F, HG=HG, HR=HR, TCH=TCH),
        grid=(T // TCH,),
        in_specs=[
            pl.BlockSpec((2, edge_index.shape[1]), lambda t: (0, 0)),
            pl.BlockSpec((F, HG), lambda t: (0, 0)),
            pl.BlockSpec((B, TCH, NF), lambda t: (0, t, 0)),
            pl.BlockSpec((3 * HR, NH), lambda t: (0, 0)),
            pl.BlockSpec((3 * HR, HR), lambda t: (0, 0)),
            pl.BlockSpec((1, NH), lambda t: (0, 0)),
            pl.BlockSpec((1, 3 * HR), lambda t: (0, 0)),
            pl.BlockSpec((1, 3 * HR), lambda t: (0, 0)),
        ],
        out_specs=[
            pl.BlockSpec((B, TCH, NH), lambda t: (0, t, 0)),
            pl.BlockSpec((B, HR), lambda t: (0, 0)),
        ],
        out_shape=[
            jax.ShapeDtypeStruct((B, T, NH), jnp.bfloat16),
            jax.ShapeDtypeStruct((B, HR), jnp.float32),
        ],
        scratch_shapes=[pltpu.VMEM((NF, NH), jnp.float32)],
        compiler_params=pltpu.CompilerParams(
            dimension_semantics=("arbitrary",)),
    )(edge_index, W_gcn, xf, W_ih, W_hh, bg, bih, bhh)

    gcn_features = gcnBT.reshape(B, T, N, HG).astype(jnp.float32)
    return gcn_features, h_final
